# baseline (device time: 65396 ns/iter reference)
import jax
import jax.numpy as jnp
from jax import lax
from jax.experimental import pallas as pl
from jax.experimental.pallas import tpu as pltpu

_MESH = pl.DeviceIdType.MESH


def kernel(x, router, W1, W2):
    T2, D = x.shape
    E_loc, _, F = W1.shape
    E = 2 * E_loc
    H = T2 // 2

    def body(x_ref, r_ref, w1_hbm, w2_hbm, out_ref,
             xs_ref, xr_ref, rr_ref, ws_ref, wr_ref,
             c1s_ref, c1r_ref, c2s_ref, c2r_ref,
             w1s_ref, w2s_ref, send_sems, recv_sems, wc_sems):
        my_x = lax.axis_index("x")
        my_y = lax.axis_index("y")
        y_nb = (my_x, 1 - my_y)
        x_nb = (1 - my_x, my_y)

        le0 = 2 * my_x
        le1 = 2 * my_x + 1

        def w_copy(hbm, le, dst, sem_i):
            return pltpu.make_async_copy(hbm.at[le], dst, wc_sems.at[sem_i])

        w_copy(w1_hbm, le0, w1s_ref.at[0], 0).start()
        w_copy(w2_hbm, le0, w2s_ref.at[0], 1).start()
        w_copy(w1_hbm, le1, w1s_ref.at[1], 2).start()
        w_copy(w2_hbm, le1, w2s_ref.at[1], 3).start()

        bar = pltpu.get_barrier_semaphore()
        pl.semaphore_signal(bar, inc=1, device_id=y_nb, device_id_type=_MESH)
        pl.semaphore_signal(bar, inc=1, device_id=x_nb, device_id_type=_MESH)
        pl.semaphore_wait(bar, 2)

        def remote(src, dst, slot, dev):
            return pltpu.make_async_remote_copy(
                src_ref=src, dst_ref=dst,
                send_sem=send_sems.at[slot], recv_sem=recv_sems.at[slot],
                device_id=dev, device_id_type=_MESH)

        r_t = remote(r_ref, rr_ref, 0, y_nb)
        x_t = [remote(xs_ref.at[pl.ds(i * H, H)], xr_ref.at[pl.ds(i * H, H)],
                      1 + i, y_nb) for i in range(2)]
        w_t = remote(ws_ref, wr_ref, 3, y_nb)
        c1 = [remote(c1s_ref.at[i], c1r_ref.at[i], 4 + i, y_nb)
              for i in range(2)]
        c2 = [remote(c2s_ref.at[i], c2r_ref.at[i], 6 + i, x_nb)
              for i in range(2)]

        r_t.start()
        xs_ref[...] = x_ref[...].astype(jnp.bfloat16)
        x_t[0].start()
        x_t[1].start()

        def f32mm(a, b):
            return lax.dot_general(a, b, (((1,), (0,)), ((), ())),
                                   precision=lax.Precision.HIGHEST,
                                   preferred_element_type=jnp.float32)

        r_t.wait_recv()
        gl = f32mm(x_ref[...], r_ref[...])
        gp = f32mm(x_ref[...], rr_ref[...])
        g = jnp.where(my_y == 0,
                      jnp.concatenate([gl, gp], axis=1),
                      jnp.concatenate([gp, gl], axis=1))
        iota = lax.broadcasted_iota(jnp.int32, g.shape, 1)
        m1 = jnp.max(g, axis=1, keepdims=True)
        i1 = jnp.min(jnp.where(g == m1, iota, E), axis=1, keepdims=True)
        g2 = jnp.where(iota == i1, -jnp.inf, g)
        m2 = jnp.max(g2, axis=1, keepdims=True)
        i2 = jnp.min(jnp.where(g2 == m2, iota, E), axis=1, keepdims=True)
        e2 = jnp.exp(m2 - m1)
        wt = (jnp.where(iota == i1, 1.0 / (1.0 + e2), 0.0)
              + jnp.where(iota == i2, e2 / (1.0 + e2), 0.0))

        def col_group(k):
            parts = [wt[:, 2 * i:2 * i + 2] for i in range(4)]
            return jnp.where(
                k == 0, parts[0],
                jnp.where(k == 1, parts[1],
                          jnp.where(k == 2, parts[2], parts[3])))

        w_my = col_group(2 * my_y + my_x)
        ws_ref[...] = col_group(2 * (1 - my_y) + my_x)
        w_t.start()

        def bf16mm(a, b):
            return lax.dot_general(a, b, (((1,), (0,)), ((), ())),
                                   preferred_element_type=jnp.float32)

        def ffn(xb, w1e, w2e, wcol):
            h = jnp.maximum(bf16mm(xb, w1e), 0.0).astype(jnp.bfloat16)
            return bf16mm(h, w2e) * wcol

        w_copy(w1_hbm, le0, w1s_ref.at[0], 0).wait()
        w_copy(w2_hbm, le0, w2s_ref.at[0], 1).wait()
        w1e0 = w1s_ref[0].astype(jnp.bfloat16)
        w2e0 = w2s_ref[0].astype(jnp.bfloat16)

        x_t[0].wait_recv()
        w_t.wait_recv()
        wrm = wr_ref[...]
        xr_a = xr_ref[pl.ds(0, H), :]
        ra_e0 = ffn(xr_a, w1e0, w2e0, wrm[0:H, 0:1])

        xl = xs_ref[...]
        acc_l = ffn(xl, w1e0, w2e0, w_my[:, 0:1])

        w_copy(w1_hbm, le1, w1s_ref.at[1], 2).wait()
        w_copy(w2_hbm, le1, w2s_ref.at[1], 3).wait()
        w1e1 = w1s_ref[1].astype(jnp.bfloat16)
        w2e1 = w2s_ref[1].astype(jnp.bfloat16)

        c1s_ref[0] = (ra_e0 + ffn(xr_a, w1e1, w2e1, wrm[0:H, 1:2])
                      ).astype(jnp.bfloat16)
        c1[0].start()

        x_t[1].wait_recv()
        xr_b = xr_ref[pl.ds(H, H), :]
        c1s_ref[1] = (ffn(xr_b, w1e0, w2e0, wrm[H:T2, 0:1])
                      + ffn(xr_b, w1e1, w2e1, wrm[H:T2, 1:2])
                      ).astype(jnp.bfloat16)
        c1[1].start()

        acc_l = acc_l + ffn(xl, w1e1, w2e1, w_my[:, 1:2])

        c1[0].wait_recv()
        q0 = acc_l[:H] + c1r_ref[0].astype(jnp.float32)
        c2s_ref[0] = q0.astype(jnp.bfloat16)
        c2[0].start()
        c1[1].wait_recv()
        q1 = acc_l[H:] + c1r_ref[1].astype(jnp.float32)
        c2s_ref[1] = q1.astype(jnp.bfloat16)
        c2[1].start()
        c2[0].wait_recv()
        out_ref[:H] = q0 + c2r_ref[0].astype(jnp.float32)
        c2[1].wait_recv()
        out_ref[H:] = q1 + c2r_ref[1].astype(jnp.float32)

        r_t.wait_send()
        w_t.wait_send()
        for t in x_t + c1 + c2:
            t.wait_send()

    return pl.pallas_call(
        body,
        out_shape=jax.ShapeDtypeStruct((T2, D), jnp.float32),
        in_specs=[
            pl.BlockSpec(memory_space=pltpu.MemorySpace.VMEM),
            pl.BlockSpec(memory_space=pltpu.MemorySpace.VMEM),
            pl.BlockSpec(memory_space=pltpu.MemorySpace.HBM),
            pl.BlockSpec(memory_space=pltpu.MemorySpace.HBM),
        ],
        out_specs=pl.BlockSpec(memory_space=pltpu.MemorySpace.VMEM),
        scratch_shapes=[
            pltpu.VMEM((T2, D), jnp.bfloat16),
            pltpu.VMEM((T2, D), jnp.bfloat16),
            pltpu.VMEM((D, E_loc), jnp.float32),
            pltpu.VMEM((T2, 2), jnp.float32),
            pltpu.VMEM((T2, 2), jnp.float32),
            pltpu.VMEM((2, H, D), jnp.bfloat16),
            pltpu.VMEM((2, H, D), jnp.bfloat16),
            pltpu.VMEM((2, H, D), jnp.bfloat16),
            pltpu.VMEM((2, H, D), jnp.bfloat16),
            pltpu.VMEM((2, D, F), jnp.float32),
            pltpu.VMEM((2, F, D), jnp.float32),
            pltpu.SemaphoreType.DMA((8,)),
            pltpu.SemaphoreType.DMA((8,)),
            pltpu.SemaphoreType.DMA((4,)),
        ],
        compiler_params=pltpu.CompilerParams(
            collective_id=0, vmem_limit_bytes=66977792),
    )(x, router, W1, W2)


# device time: 64323 ns/iter; 1.0167x vs baseline; 1.0167x over previous
import os

import jax
import jax.numpy as jnp
from jax import lax
from jax.experimental import pallas as pl
from jax.experimental.pallas import tpu as pltpu

_MESH = pl.DeviceIdType.MESH


def kernel(x, router, W1, W2):
    T2, D = x.shape
    E_loc, _, F = W1.shape
    E = 2 * E_loc
    H = T2 // 2

    def body(x_ref, r_ref, w1_hbm, w2_hbm, out_ref,
             xs_ref, xr_ref, rr_ref, ws_ref, wr_ref,
             c1s_ref, c1r_ref, c2s_ref, c2r_ref,
             w1s_ref, w2s_ref, send_sems, recv_sems, wc_sems):
        my_x = lax.axis_index("x")
        my_y = lax.axis_index("y")
        y_nb = (my_x, 1 - my_y)
        x_nb = (1 - my_x, my_y)

        le0 = 2 * my_x
        le1 = 2 * my_x + 1

        def w_copy(hbm, le, dst, sem_i):
            return pltpu.make_async_copy(hbm.at[le], dst, wc_sems.at[sem_i])

        w_copy(w1_hbm, le0, w1s_ref.at[0], 0).start()
        w_copy(w2_hbm, le0, w2s_ref.at[0], 1).start()
        w_copy(w1_hbm, le1, w1s_ref.at[1], 2).start()
        w_copy(w2_hbm, le1, w2s_ref.at[1], 3).start()

        if os.environ.get("KERNEL_AUTOBARRIER", "0") != "1":
            bar = pltpu.get_barrier_semaphore()
            pl.semaphore_signal(bar, inc=1, device_id=y_nb,
                                device_id_type=_MESH)
            pl.semaphore_signal(bar, inc=1, device_id=x_nb,
                                device_id_type=_MESH)
            pl.semaphore_wait(bar, 2)

        def remote(src, dst, slot, dev):
            return pltpu.make_async_remote_copy(
                src_ref=src, dst_ref=dst,
                send_sem=send_sems.at[slot], recv_sem=recv_sems.at[slot],
                device_id=dev, device_id_type=_MESH)

        r_t = remote(r_ref, rr_ref, 0, y_nb)
        x_t = [remote(xs_ref.at[pl.ds(i * H, H)], xr_ref.at[pl.ds(i * H, H)],
                      1 + i, y_nb) for i in range(2)]
        w_t = remote(ws_ref, wr_ref, 3, y_nb)
        c1 = [remote(c1s_ref.at[i], c1r_ref.at[i], 4 + i, y_nb)
              for i in range(2)]
        c2 = [remote(c2s_ref.at[i], c2r_ref.at[i], 6 + i, x_nb)
              for i in range(2)]

        r_t.start()
        xs_ref[...] = x_ref[...].astype(jnp.bfloat16)
        x_t[0].start()
        x_t[1].start()

        def f32mm(a, b):
            return lax.dot_general(a, b, (((1,), (0,)), ((), ())),
                                   precision=lax.Precision.HIGHEST,
                                   preferred_element_type=jnp.float32)

        r_t.wait_recv()
        gl = f32mm(x_ref[...], r_ref[...])
        gp = f32mm(x_ref[...], rr_ref[...])
        g = jnp.where(my_y == 0,
                      jnp.concatenate([gl, gp], axis=1),
                      jnp.concatenate([gp, gl], axis=1))
        iota = lax.broadcasted_iota(jnp.int32, g.shape, 1)
        m1 = jnp.max(g, axis=1, keepdims=True)
        i1 = jnp.min(jnp.where(g == m1, iota, E), axis=1, keepdims=True)
        g2 = jnp.where(iota == i1, -jnp.inf, g)
        m2 = jnp.max(g2, axis=1, keepdims=True)
        i2 = jnp.min(jnp.where(g2 == m2, iota, E), axis=1, keepdims=True)
        e2 = jnp.exp(m2 - m1)
        wt = (jnp.where(iota == i1, 1.0 / (1.0 + e2), 0.0)
              + jnp.where(iota == i2, e2 / (1.0 + e2), 0.0))

        def col_group(k):
            parts = [wt[:, 2 * i:2 * i + 2] for i in range(4)]
            return jnp.where(
                k == 0, parts[0],
                jnp.where(k == 1, parts[1],
                          jnp.where(k == 2, parts[2], parts[3])))

        w_my = col_group(2 * my_y + my_x)
        ws_ref[...] = col_group(2 * (1 - my_y) + my_x)
        w_t.start()

        def bf16mm(a, b):
            return lax.dot_general(a, b, (((1,), (0,)), ((), ())),
                                   preferred_element_type=jnp.float32)

        def ffn(xb, w1e, w2e, wcol):
            h = jnp.maximum(bf16mm(xb, w1e), 0.0).astype(jnp.bfloat16)
            return bf16mm(h, w2e) * wcol

        w_copy(w1_hbm, le0, w1s_ref.at[0], 0).wait()
        w_copy(w2_hbm, le0, w2s_ref.at[0], 1).wait()
        w1e0 = w1s_ref[0].astype(jnp.bfloat16)
        w2e0 = w2s_ref[0].astype(jnp.bfloat16)

        xl = xs_ref[...]
        acc_l = ffn(xl, w1e0, w2e0, w_my[:, 0:1])

        x_t[0].wait_recv()
        w_t.wait_recv()
        wrm = wr_ref[...]
        xr_a = xr_ref[pl.ds(0, H), :]
        ra_e0 = ffn(xr_a, w1e0, w2e0, wrm[0:H, 0:1])

        w_copy(w1_hbm, le1, w1s_ref.at[1], 2).wait()
        w_copy(w2_hbm, le1, w2s_ref.at[1], 3).wait()
        w1e1 = w1s_ref[1].astype(jnp.bfloat16)
        w2e1 = w2s_ref[1].astype(jnp.bfloat16)

        c1s_ref[0] = (ra_e0 + ffn(xr_a, w1e1, w2e1, wrm[0:H, 1:2])
                      ).astype(jnp.bfloat16)
        c1[0].start()

        x_t[1].wait_recv()
        xr_b = xr_ref[pl.ds(H, H), :]
        c1s_ref[1] = (ffn(xr_b, w1e0, w2e0, wrm[H:T2, 0:1])
                      + ffn(xr_b, w1e1, w2e1, wrm[H:T2, 1:2])
                      ).astype(jnp.bfloat16)
        c1[1].start()

        acc_l = acc_l + ffn(xl, w1e1, w2e1, w_my[:, 1:2])

        c1[0].wait_recv()
        q0 = acc_l[:H] + c1r_ref[0].astype(jnp.float32)
        c2s_ref[0] = q0.astype(jnp.bfloat16)
        c2[0].start()
        c1[1].wait_recv()
        q1 = acc_l[H:] + c1r_ref[1].astype(jnp.float32)
        c2s_ref[1] = q1.astype(jnp.bfloat16)
        c2[1].start()
        c2[0].wait_recv()
        out_ref[:H] = q0 + c2r_ref[0].astype(jnp.float32)
        c2[1].wait_recv()
        out_ref[H:] = q1 + c2r_ref[1].astype(jnp.float32)

        r_t.wait_send()
        w_t.wait_send()
        for t in x_t + c1 + c2:
            t.wait_send()

    return pl.pallas_call(
        body,
        out_shape=jax.ShapeDtypeStruct((T2, D), jnp.float32),
        in_specs=[
            pl.BlockSpec(memory_space=pltpu.MemorySpace.VMEM),
            pl.BlockSpec(memory_space=pltpu.MemorySpace.VMEM),
            pl.BlockSpec(memory_space=pltpu.MemorySpace.HBM),
            pl.BlockSpec(memory_space=pltpu.MemorySpace.HBM),
        ],
        out_specs=pl.BlockSpec(memory_space=pltpu.MemorySpace.VMEM),
        scratch_shapes=[
            pltpu.VMEM((T2, D), jnp.bfloat16),
            pltpu.VMEM((T2, D), jnp.bfloat16),
            pltpu.VMEM((D, E_loc), jnp.float32),
            pltpu.VMEM((T2, 2), jnp.float32),
            pltpu.VMEM((T2, 2), jnp.float32),
            pltpu.VMEM((2, H, D), jnp.bfloat16),
            pltpu.VMEM((2, H, D), jnp.bfloat16),
            pltpu.VMEM((2, H, D), jnp.bfloat16),
            pltpu.VMEM((2, H, D), jnp.bfloat16),
            pltpu.VMEM((2, D, F), jnp.float32),
            pltpu.VMEM((2, F, D), jnp.float32),
            pltpu.SemaphoreType.DMA((8,)),
            pltpu.SemaphoreType.DMA((8,)),
            pltpu.SemaphoreType.DMA((4,)),
        ],
        compiler_params=(
            pltpu.CompilerParams(vmem_limit_bytes=66977792)
            if os.environ.get("KERNEL_AUTOBARRIER", "0") == "1"
            else pltpu.CompilerParams(
                collective_id=0, vmem_limit_bytes=66977792)),
    )(x, router, W1, W2)


# device time: 62017 ns/iter; 1.0545x vs baseline; 1.0372x over previous
import jax
import jax.numpy as jnp
from jax import lax
from jax.experimental import pallas as pl
from jax.experimental.pallas import tpu as pltpu

_MESH = pl.DeviceIdType.MESH


def kernel(x, router, W1, W2):
    T2, D = x.shape
    E_loc, _, F = W1.shape
    E = 2 * E_loc
    H = T2 // 2

    def body(x_ref, r_ref, w1_hbm, w2_hbm, out_ref,
             xs_ref, xr_ref, rr_ref, ws_ref, wr_ref,
             c1s_ref, c1r_ref, c2s_ref, c2r_ref,
             w1s_ref, w2s_ref, send_sems, recv_sems, wc_sems):
        my_x = lax.axis_index("x")
        my_y = lax.axis_index("y")
        y_nb = (my_x, 1 - my_y)
        x_nb = (1 - my_x, my_y)

        le0 = 2 * my_x
        le1 = 2 * my_x + 1

        def w_copy(hbm, le, dst, sem_i):
            return pltpu.make_async_copy(hbm.at[le], dst, wc_sems.at[sem_i])

        w_copy(w1_hbm, le0, w1s_ref.at[0], 0).start()
        w_copy(w2_hbm, le0, w2s_ref.at[0], 1).start()
        w_copy(w1_hbm, le1, w1s_ref.at[1], 2).start()
        w_copy(w2_hbm, le1, w2s_ref.at[1], 3).start()

        bar = pltpu.get_barrier_semaphore()
        pl.semaphore_signal(bar, inc=1, device_id=y_nb, device_id_type=_MESH)
        pl.semaphore_signal(bar, inc=1, device_id=x_nb, device_id_type=_MESH)
        pl.semaphore_wait(bar, 2)

        def remote(src, dst, slot, dev):
            return pltpu.make_async_remote_copy(
                src_ref=src, dst_ref=dst,
                send_sem=send_sems.at[slot], recv_sem=recv_sems.at[slot],
                device_id=dev, device_id_type=_MESH)

        r_t = remote(r_ref, rr_ref, 0, y_nb)
        x_t = [remote(xs_ref.at[pl.ds(i * H, H)], xr_ref.at[pl.ds(i * H, H)],
                      1 + i, y_nb) for i in range(2)]
        w_t = remote(ws_ref, wr_ref, 3, y_nb)
        c1 = [remote(c1s_ref.at[i], c1r_ref.at[i], 4 + i, y_nb)
              for i in range(2)]
        c2 = [remote(c2s_ref.at[i], c2r_ref.at[i], 6 + i, x_nb)
              for i in range(2)]

        r_t.start()
        xs_ref[...] = x_ref[...].astype(jnp.bfloat16)
        x_t[0].start()
        x_t[1].start()

        def f32mm(a, b):
            return lax.dot_general(a, b, (((1,), (0,)), ((), ())),
                                   precision=lax.Precision.HIGHEST,
                                   preferred_element_type=jnp.float32)

        r_t.wait_recv()
        gl = f32mm(x_ref[...], r_ref[...])
        gp = f32mm(x_ref[...], rr_ref[...])
        g = jnp.where(my_y == 0,
                      jnp.concatenate([gl, gp], axis=1),
                      jnp.concatenate([gp, gl], axis=1))
        iota = lax.broadcasted_iota(jnp.int32, g.shape, 1)
        m1 = jnp.max(g, axis=1, keepdims=True)
        i1 = jnp.min(jnp.where(g == m1, iota, E), axis=1, keepdims=True)
        g2 = jnp.where(iota == i1, -jnp.inf, g)
        m2 = jnp.max(g2, axis=1, keepdims=True)
        i2 = jnp.min(jnp.where(g2 == m2, iota, E), axis=1, keepdims=True)
        e2 = jnp.exp(m2 - m1)
        wt = (jnp.where(iota == i1, 1.0 / (1.0 + e2), 0.0)
              + jnp.where(iota == i2, e2 / (1.0 + e2), 0.0))

        def col_group(k):
            parts = [wt[:, 2 * i:2 * i + 2] for i in range(4)]
            return jnp.where(
                k == 0, parts[0],
                jnp.where(k == 1, parts[1],
                          jnp.where(k == 2, parts[2], parts[3])))

        w_my = col_group(2 * my_y + my_x)
        ws_ref[...] = col_group(2 * (1 - my_y) + my_x)
        w_t.start()

        def bf16mm(a, b):
            return lax.dot_general(a, b, (((1,), (0,)), ((), ())),
                                   preferred_element_type=jnp.float32)

        def ffn(xb, w1e, w2e, wcol):
            h = jnp.maximum(bf16mm(xb, w1e), 0.0).astype(jnp.bfloat16)
            return bf16mm(h, w2e) * wcol

        w_copy(w1_hbm, le0, w1s_ref.at[0], 0).wait()
        w_copy(w2_hbm, le0, w2s_ref.at[0], 1).wait()
        w1e0 = w1s_ref[0].astype(jnp.bfloat16)
        w2e0 = w2s_ref[0].astype(jnp.bfloat16)

        xl = xs_ref[...]
        acc_l = ffn(xl, w1e0, w2e0, w_my[:, 0:1])

        x_t[0].wait_recv()
        w_t.wait_recv()
        wrm = wr_ref[...]
        xr_a = xr_ref[pl.ds(0, H), :]
        ra_e0 = ffn(xr_a, w1e0, w2e0, wrm[0:H, 0:1])

        w_copy(w1_hbm, le1, w1s_ref.at[1], 2).wait()
        w_copy(w2_hbm, le1, w2s_ref.at[1], 3).wait()
        w1e1 = w1s_ref[1].astype(jnp.bfloat16)
        w2e1 = w2s_ref[1].astype(jnp.bfloat16)

        c1s_ref[0] = (ra_e0 + ffn(xr_a, w1e1, w2e1, wrm[0:H, 1:2])
                      ).astype(jnp.bfloat16)
        c1[0].start()

        x_t[1].wait_recv()
        xr_b = xr_ref[pl.ds(H, H), :]
        c1s_ref[1] = (ffn(xr_b, w1e0, w2e0, wrm[H:T2, 0:1])
                      + ffn(xr_b, w1e1, w2e1, wrm[H:T2, 1:2])
                      ).astype(jnp.bfloat16)
        c1[1].start()

        acc_l = acc_l + ffn(xl, w1e1, w2e1, w_my[:, 1:2])

        c1[0].wait_recv()
        q0 = acc_l[:H] + c1r_ref[0].astype(jnp.float32)
        c2s_ref[0] = q0.astype(jnp.bfloat16)
        c2[0].start()
        c1[1].wait_recv()
        q1 = acc_l[H:] + c1r_ref[1].astype(jnp.float32)
        c2s_ref[1] = q1.astype(jnp.bfloat16)
        c2[1].start()
        c2[0].wait_recv()
        out_ref[:H] = q0 + c2r_ref[0].astype(jnp.float32)
        c2[1].wait_recv()
        out_ref[H:] = q1 + c2r_ref[1].astype(jnp.float32)

        r_t.wait_send()
        w_t.wait_send()
        for t in x_t + c1 + c2:
            t.wait_send()

    return pl.pallas_call(
        body,
        out_shape=jax.ShapeDtypeStruct((T2, D), jnp.float32),
        in_specs=[
            pl.BlockSpec(memory_space=pltpu.MemorySpace.VMEM),
            pl.BlockSpec(memory_space=pltpu.MemorySpace.VMEM),
            pl.BlockSpec(memory_space=pltpu.MemorySpace.HBM),
            pl.BlockSpec(memory_space=pltpu.MemorySpace.HBM),
        ],
        out_specs=pl.BlockSpec(memory_space=pltpu.MemorySpace.VMEM),
        scratch_shapes=[
            pltpu.VMEM((T2, D), jnp.bfloat16),
            pltpu.VMEM((T2, D), jnp.bfloat16),
            pltpu.VMEM((D, E_loc), jnp.float32),
            pltpu.VMEM((T2, 2), jnp.float32),
            pltpu.VMEM((T2, 2), jnp.float32),
            pltpu.VMEM((2, H, D), jnp.bfloat16),
            pltpu.VMEM((2, H, D), jnp.bfloat16),
            pltpu.VMEM((2, H, D), jnp.bfloat16),
            pltpu.VMEM((2, H, D), jnp.bfloat16),
            pltpu.VMEM((2, D, F), jnp.float32),
            pltpu.VMEM((2, F, D), jnp.float32),
            pltpu.SemaphoreType.DMA((8,)),
            pltpu.SemaphoreType.DMA((8,)),
            pltpu.SemaphoreType.DMA((4,)),
        ],
        compiler_params=pltpu.CompilerParams(
            collective_id=0, vmem_limit_bytes=66977792),
    )(x, router, W1, W2)
